# Initial kernel scaffold; baseline (speedup 1.0000x reference)
import jax
import jax.numpy as jnp
from jax.experimental import pallas as pl


def kernel(x, level, fm):
    raise NotImplementedError("write your pallas kernel here")


# speedup vs baseline: 189.2331x; 189.2331x over previous
import functools

import jax
import jax.numpy as jnp
from jax import lax
from jax.experimental import pallas as pl
from jax.experimental.pallas import tpu as pltpu
from jax.experimental.pallas import tpu_sc as plsc

_N_LEVELS = 8
_PLANE = 512
_FDIM = 16
_TOTAL = sum(((_PLANE >> l) ** 2 for l in range(_N_LEVELS)))

_NC = 2
_NS = 16
_NW = _NC * _NS
_C = 128
_G = 24


def _pool_body(in_ref, out_ref):
    xb = in_ref[0]
    r = xb[0] + xb[1]
    w = r.shape[0]
    rr = r.reshape(w // 2, 2, _FDIM)
    out_ref[0, 0] = (rr[:, 0, :] + rr[:, 1, :]) * 0.25


def _avg_pool2(t):
    h = t.shape[1]
    w = t.shape[2]
    return pl.pallas_call(
        _pool_body,
        grid=(3, h // 2),
        in_specs=[pl.BlockSpec((1, 2, w, _FDIM), lambda p, i: (p, i, 0, 0))],
        out_specs=pl.BlockSpec((1, 1, w // 2, _FDIM), lambda p, i: (p, i, 0, 0)),
        out_shape=jax.ShapeDtypeStruct((3, h // 2, w // 2, _FDIM), t.dtype),
    )(t)


def _sc_body(x0h, x1h, x2h, lvh, ath, outh, crd, idxb, wb, rows, outb, sem):
    wid = lax.axis_index("s") * _NC + lax.axis_index("c")
    npts = x0h.shape[0]
    per_w = npts // _NW

    def chunk_body(i, carry):
        base = wid * per_w + i * _C
        pltpu.sync_copy(x0h.at[pl.ds(base, _C)], crd.at[0])
        pltpu.sync_copy(x1h.at[pl.ds(base, _C)], crd.at[1])
        pltpu.sync_copy(x2h.at[pl.ds(base, _C)], crd.at[2])
        pltpu.sync_copy(lvh.at[pl.ds(base, _C)], crd.at[3])

        for v in range(_C // 16):
            o = v * 16
            a0 = crd[0, pl.ds(o, 16)]
            a1 = crd[1, pl.ds(o, 16)]
            a2 = crd[2, pl.ds(o, 16)]
            lv = crd[3, pl.ds(o, 16)]
            lvc = jnp.clip(lv, 0.0, float(_N_LEVELS - 1))
            l0i = lvc.astype(jnp.int32)
            t = lvc - l0i.astype(jnp.float32)
            l1i = jnp.minimum(l0i + 1, _N_LEVELS - 1)
            one = jnp.full((16,), 1, jnp.int32)
            for p in range(3):
                u = (a1, a0, a0)[p]
                vv = (a2, a2, a1)[p]
                for s in range(2):
                    li = l0i if s == 0 else l1i
                    wl = (1.0 - t) if s == 0 else t
                    wi = lax.shift_left(one, 9 - li)
                    wf = wi.astype(jnp.float32)
                    fx = u * wf - 0.5
                    ix = (fx + 1.0).astype(jnp.int32) - 1
                    tx = fx - ix.astype(jnp.float32)
                    fy = vv * wf - 0.5
                    iy = (fy + 1.0).astype(jnp.int32) - 1
                    ty = fy - iy.astype(jnp.float32)
                    wm1 = wi - 1
                    zero = jnp.zeros((16,), jnp.int32)
                    x0c = jnp.clip(ix, zero, wm1)
                    x1c = jnp.clip(ix + 1, zero, wm1)
                    y0c = jnp.clip(iy, zero, wm1)
                    y1c = jnp.clip(iy + 1, zero, wm1)
                    off = lax.div(1048576 - lax.shift_left(one, 20 - 2 * li), 3) + p * _TOTAL
                    r0 = off + y0c * wi
                    r1 = off + y1c * wi
                    g = p * 8 + s * 4
                    idxb[g + 0, pl.ds(o, 16)] = r0 + x0c
                    idxb[g + 1, pl.ds(o, 16)] = r0 + x1c
                    idxb[g + 2, pl.ds(o, 16)] = r1 + x0c
                    idxb[g + 3, pl.ds(o, 16)] = r1 + x1c
                    omtx = 1.0 - tx
                    omty = 1.0 - ty
                    wb[g + 0, pl.ds(o, 16)] = wl * omtx * omty
                    wb[g + 1, pl.ds(o, 16)] = wl * tx * omty
                    wb[g + 2, pl.ds(o, 16)] = wl * omtx * ty
                    wb[g + 3, pl.ds(o, 16)] = wl * tx * ty

        copies = [
            pltpu.async_copy(ath.at[idxb.at[q]], rows.at[q], sem)
            for q in range(_G)
        ]
        for c in copies:
            c.wait()

        def grp_body(jg, carry2):
            o = pl.multiple_of(jg * 16, 16)
            wvs = [wb[g, pl.ds(o, 16)] for g in range(_G)]
            for k in range(16):
                j = o + k
                for p in range(3):
                    acc = jnp.zeros((16,), jnp.float32)
                    for kk in range(8):
                        g = p * 8 + kk
                        acc = acc + rows[g, j] * wvs[g][k]
                    outb[j, pl.ds(p * 16, 16)] = acc
            return carry2

        lax.fori_loop(0, _C // 16, grp_body, 0)
        pltpu.sync_copy(outb, outh.at[pl.ds(base, _C)])
        return carry

    lax.fori_loop(0, per_w // _C, chunk_body, 0)


def kernel(x, level, fm):
    n = x.shape[0]
    mips = [fm]
    for _ in range(_N_LEVELS - 1):
        mips.append(_avg_pool2(mips[-1]))
    atlas = jnp.concatenate(
        [m.reshape(3, -1, _FDIM) for m in mips], axis=1
    ).reshape(3 * _TOTAL, _FDIM)

    x0 = x[:, 0]
    x1 = x[:, 1]
    x2 = x[:, 2]
    lvl = level[:, 0]

    mesh = plsc.VectorSubcoreMesh(core_axis_name="c", subcore_axis_name="s")
    sc = functools.partial(
        pl.kernel,
        mesh=mesh,
        out_type=jax.ShapeDtypeStruct((n, 3 * _FDIM), jnp.float32),
        scratch_types=[
            pltpu.VMEM((4, _C), jnp.float32),
            pltpu.VMEM((_G, _C), jnp.int32),
            pltpu.VMEM((_G, _C), jnp.float32),
            pltpu.VMEM((_G, _C, _FDIM), jnp.float32),
            pltpu.VMEM((_C, 3 * _FDIM), jnp.float32),
            pltpu.SemaphoreType.DMA,
        ],
        compiler_params=pltpu.CompilerParams(use_tc_tiling_on_sc=False),
    )(_sc_body)
    return sc(x0, x1, x2, lvl, atlas)
